# initial kernel scaffold (unmeasured)
import jax
import jax.numpy as jnp
from jax import lax
from jax.experimental import pallas as pl
from jax.experimental.pallas import tpu as pltpu

N_DEV = 4
M_PER = 1024
K = 4096
N_PER = 2048


def kernel(x, w_mat, scale_x, scale_w):
    my = lax.axis_index("i")
    x8 = x.astype(jnp.float8_e4m3fn)
    w8 = lax.dynamic_slice(w_mat, (0, my * N_PER), (K, N_PER)).astype(
        jnp.float8_e5m2
    )
    s = (scale_x * scale_w).reshape(1, 1).astype(jnp.float32)

    def body(x_ref, w_ref, s_ref, out_ref, comm_ref, send_sems, recv_sems):
        my_pos = lax.axis_index("i")
        left = lax.rem(my_pos + (N_DEV - 1), N_DEV)
        right = lax.rem(my_pos + 1, N_DEV)

        barrier_sem = pltpu.get_barrier_semaphore()
        for nbr in [left, right]:
            pl.semaphore_signal(
                barrier_sem, inc=1,
                device_id=(nbr,), device_id_type=pl.DeviceIdType.MESH,
            )
        pl.semaphore_wait(barrier_sem, 2)

        scale = s_ref[0, 0]

        comm_ref[0, :, :] = x_ref[:, :]
        acc = jnp.dot(x_ref[:, :], w_ref[:, :],
                      preferred_element_type=jnp.float32)
        out_ref[pl.ds(my_pos * M_PER, M_PER), :] = acc * scale

        for h in range(N_DEV - 1):
            send_slot = h % 2
            recv_slot = (h + 1) % 2
            rdma = pltpu.make_async_remote_copy(
                src_ref=comm_ref.at[send_slot],
                dst_ref=comm_ref.at[recv_slot],
                send_sem=send_sems.at[send_slot],
                recv_sem=recv_sems.at[recv_slot],
                device_id=(right,),
                device_id_type=pl.DeviceIdType.MESH,
            )
            rdma.start()
            rdma.wait()

            origin = lax.rem(my_pos + (N_DEV - 1 - h), N_DEV)
            acc = jnp.dot(comm_ref[recv_slot, :, :], w_ref[:, :],
                          preferred_element_type=jnp.float32)
            out_ref[pl.ds(origin * M_PER, M_PER), :] = acc * scale

    return pl.pallas_call(
        body,
        out_shape=jax.ShapeDtypeStruct((N_DEV * M_PER, N_PER), jnp.float32),
        in_specs=[
            pl.BlockSpec(memory_space=pltpu.VMEM),
            pl.BlockSpec(memory_space=pltpu.VMEM),
            pl.BlockSpec(memory_space=pltpu.SMEM),
        ],
        out_specs=pl.BlockSpec(memory_space=pltpu.VMEM),
        scratch_shapes=[
            pltpu.VMEM((2, M_PER, K), jnp.float8_e4m3fn),
            pltpu.SemaphoreType.DMA((2,)),
            pltpu.SemaphoreType.DMA((2,)),
        ],
        compiler_params=pltpu.CompilerParams(collective_id=0),
    )(x8, w8, s)


# baseline (device time: 245747 ns/iter reference)
import jax
import jax.numpy as jnp
from jax import lax
from jax.experimental import pallas as pl
from jax.experimental.pallas import tpu as pltpu

N_DEV = 4
M_PER = 1024
K = 4096
N_PER = 2048


def kernel(x, w_mat, scale_x, scale_w):
    my = lax.axis_index("i")
    x8 = x.astype(jnp.float8_e4m3fn)
    w8 = lax.dynamic_slice(w_mat, (0, my * N_PER), (K, N_PER)).astype(
        jnp.float8_e5m2
    )
    s = (scale_x * scale_w).reshape(1, 1).astype(jnp.float32)

    def body(x_ref, w_ref, s_ref, out_ref, comm_ref, send_sems, recv_sems):
        my_pos = lax.axis_index("i")
        left = lax.rem(my_pos + (N_DEV - 1), N_DEV)
        right = lax.rem(my_pos + 1, N_DEV)

        barrier_sem = pltpu.get_barrier_semaphore()
        for nbr in [left, right]:
            pl.semaphore_signal(
                barrier_sem, inc=1,
                device_id=(nbr,), device_id_type=pl.DeviceIdType.MESH,
            )
        pl.semaphore_wait(barrier_sem, 2)

        scale = s_ref[0, 0]

        comm_ref[0, :, :] = x_ref[:, :]
        acc = jnp.dot(x_ref[:, :], w_ref[:, :],
                      preferred_element_type=jnp.float32)
        out_ref[pl.ds(my_pos * M_PER, M_PER), :] = acc * scale

        for h in range(N_DEV - 1):
            send_slot = h % 2
            recv_slot = (h + 1) % 2
            rdma = pltpu.make_async_remote_copy(
                src_ref=comm_ref.at[send_slot],
                dst_ref=comm_ref.at[recv_slot],
                send_sem=send_sems.at[send_slot],
                recv_sem=recv_sems.at[recv_slot],
                device_id=(right,),
                device_id_type=pl.DeviceIdType.MESH,
            )
            rdma.start()
            rdma.wait()

            origin = lax.rem(my_pos + (N_DEV - 1 - h), N_DEV)
            acc = jnp.dot(comm_ref[recv_slot, :, :], w_ref[:, :],
                          preferred_element_type=jnp.float32)
            out_ref[pl.ds(origin * M_PER, M_PER), :] = acc * scale

    return pl.pallas_call(
        body,
        out_shape=jax.ShapeDtypeStruct((N_DEV * M_PER, N_PER), jnp.float32),
        in_specs=[
            pl.BlockSpec(memory_space=pltpu.VMEM),
            pl.BlockSpec(memory_space=pltpu.VMEM),
            pl.BlockSpec(memory_space=pltpu.SMEM),
        ],
        out_specs=pl.BlockSpec(memory_space=pltpu.VMEM),
        scratch_shapes=[
            pltpu.VMEM((2, M_PER, K), jnp.float8_e4m3fn),
            pltpu.SemaphoreType.DMA((2,)),
            pltpu.SemaphoreType.DMA((2,)),
        ],
        compiler_params=pltpu.CompilerParams(
            collective_id=0,
            vmem_limit_bytes=60 * 1024 * 1024,
        ),
    )(x8, w8, s)


# device time: 111812 ns/iter; 2.1979x vs baseline; 2.1979x over previous
import jax
import jax.numpy as jnp
from jax import lax
from jax.experimental import pallas as pl
from jax.experimental.pallas import tpu as pltpu

N_DEV = 4
M_PER = 1024
M_HALF = M_PER // 2
K = 4096
N_PER = 2048
N_HOP = N_DEV - 1
WCH = 256
N_WCH = N_PER // WCH


def kernel(x, w_mat, scale_x, scale_w):
    s = (scale_x * scale_w).reshape(1, 1).astype(jnp.float32)

    def body(x_hbm, w_hbm, s_ref, out_ref,
             cw_ref, ccw_ref, w8_ref, wstage, xstage,
             w_sems, x_sem,
             cw_send, cw_recv, ccw_send, ccw_recv):
        my_pos = lax.axis_index("i")
        left = lax.rem(my_pos + (N_DEV - 1), N_DEV)
        right = lax.rem(my_pos + 1, N_DEV)
        col0 = my_pos * N_PER

        def w_copy(c):
            return pltpu.make_async_copy(
                w_hbm.at[:, pl.ds(col0 + c * WCH, WCH)],
                wstage.at[c % 2],
                w_sems.at[c % 2],
            )

        x_top = pltpu.make_async_copy(
            x_hbm.at[pl.ds(0, M_HALF)], xstage, x_sem)
        x_bot = pltpu.make_async_copy(
            x_hbm.at[pl.ds(M_HALF, M_HALF)], xstage, x_sem)

        x_top.start()
        w_copy(0).start()
        w_copy(1).start()

        barrier_sem = pltpu.get_barrier_semaphore()
        for nbr in [left, right]:
            pl.semaphore_signal(
                barrier_sem, inc=1,
                device_id=(nbr,), device_id_type=pl.DeviceIdType.MESH,
            )
        pl.semaphore_wait(barrier_sem, 2)

        scale = s_ref[0, 0]

        def make_rdma(h):
            cw = pltpu.make_async_remote_copy(
                src_ref=cw_ref.at[h],
                dst_ref=cw_ref.at[h + 1],
                send_sem=cw_send.at[h],
                recv_sem=cw_recv.at[h],
                device_id=(right,),
                device_id_type=pl.DeviceIdType.MESH,
            )
            ccw = pltpu.make_async_remote_copy(
                src_ref=ccw_ref.at[h],
                dst_ref=ccw_ref.at[h + 1],
                send_sem=ccw_send.at[h],
                recv_sem=ccw_recv.at[h],
                device_id=(left,),
                device_id_type=pl.DeviceIdType.MESH,
            )
            return cw, ccw

        rdmas = [make_rdma(h) for h in range(N_HOP)]

        x_top.wait()
        cw_ref[0, :, :] = xstage[:, :].astype(jnp.float8_e4m3fn)
        rdmas[0][0].start()
        x_bot.start()
        x_bot.wait()
        ccw_ref[0, :, :] = xstage[:, :].astype(jnp.float8_e4m3fn)
        rdmas[0][1].start()

        for c in range(N_WCH):
            w_copy(c).wait()
            if c + 2 < N_WCH:
                w_copy(c + 2).start()
            w8_ref[:, pl.ds(c * WCH, WCH)] = wstage[c % 2].astype(
                jnp.float8_e5m2)

        acc_t = jnp.dot(cw_ref[0, :, :], w8_ref[:, :],
                        preferred_element_type=jnp.float32)
        out_ref[pl.ds(my_pos * M_PER, M_HALF), :] = (
            acc_t * scale).astype(jnp.bfloat16)
        acc_b = jnp.dot(ccw_ref[0, :, :], w8_ref[:, :],
                        preferred_element_type=jnp.float32)
        out_ref[pl.ds(my_pos * M_PER + M_HALF, M_HALF), :] = (
            acc_b * scale).astype(jnp.bfloat16)

        for h in range(N_HOP):
            rdmas[h][0].wait_recv()
            if h + 1 < N_HOP:
                rdmas[h + 1][0].start()
            rdmas[h][1].wait_recv()
            if h + 1 < N_HOP:
                rdmas[h + 1][1].start()

            o_cw = lax.rem(my_pos + (N_DEV - 1 - h), N_DEV)
            o_ccw = lax.rem(my_pos + h + 1, N_DEV)
            acc_t = jnp.dot(cw_ref[h + 1, :, :], w8_ref[:, :],
                            preferred_element_type=jnp.float32)
            out_ref[pl.ds(o_cw * M_PER, M_HALF), :] = (
                acc_t * scale).astype(jnp.bfloat16)
            acc_b = jnp.dot(ccw_ref[h + 1, :, :], w8_ref[:, :],
                            preferred_element_type=jnp.float32)
            out_ref[pl.ds(o_ccw * M_PER + M_HALF, M_HALF), :] = (
                acc_b * scale).astype(jnp.bfloat16)

        for h in range(N_HOP):
            rdmas[h][0].wait_send()
            rdmas[h][1].wait_send()

    return pl.pallas_call(
        body,
        out_shape=jax.ShapeDtypeStruct((N_DEV * M_PER, N_PER), jnp.bfloat16),
        in_specs=[
            pl.BlockSpec(memory_space=pltpu.MemorySpace.HBM),
            pl.BlockSpec(memory_space=pltpu.MemorySpace.HBM),
            pl.BlockSpec(memory_space=pltpu.MemorySpace.SMEM),
        ],
        out_specs=pl.BlockSpec(memory_space=pltpu.MemorySpace.VMEM),
        scratch_shapes=[
            pltpu.VMEM((N_DEV, M_HALF, K), jnp.float8_e4m3fn),
            pltpu.VMEM((N_DEV, M_HALF, K), jnp.float8_e4m3fn),
            pltpu.VMEM((K, N_PER), jnp.float8_e5m2),
            pltpu.VMEM((2, K, WCH), jnp.float32),
            pltpu.VMEM((M_HALF, K), jnp.float32),
            pltpu.SemaphoreType.DMA((2,)),
            pltpu.SemaphoreType.DMA,
            pltpu.SemaphoreType.DMA((N_HOP,)),
            pltpu.SemaphoreType.DMA((N_HOP,)),
            pltpu.SemaphoreType.DMA((N_HOP,)),
            pltpu.SemaphoreType.DMA((N_HOP,)),
        ],
        compiler_params=pltpu.CompilerParams(
            collective_id=0,
            vmem_limit_bytes=60 * 1024 * 1024,
        ),
    )(x, w_mat, s)
